# baseline (speedup 1.0000x reference)
import dataclasses
import functools

import jax
import jax.numpy as jnp
from jax import lax
from jax.experimental import pallas as pl
from jax.experimental.pallas import tpu as pltpu
from jax.experimental.pallas import tpu_sc as plsc

N = 10000
E = 320000
D = 128
ED = 16
K = 32

NC = 2
NS = 16
NW = NC * NS

PER_W = 10240
TOT = NW * PER_W
NP = TOT // K

CE = 256

CH_C0 = 60
CH_C1 = 20
R0 = CH_C0 * CE
R1 = CH_C1 * CE
RP = R0 + R1

_mesh = plsc.VectorSubcoreMesh(core_axis_name="c", subcore_axis_name="s")

_no_layout = pltpu.CompilerParams()
if "needs_layout_passes" in pltpu.CompilerParams.__dataclass_fields__:
    _no_layout = dataclasses.replace(_no_layout, needs_layout_passes=False)


def _wid():
    return lax.axis_index("s") * NC + lax.axis_index("c")


def _make_row_gather(dtype, ch0=CH_C0, ch1=CH_C1, tot=TOT):
    r0 = ch0 * CE
    rp = (ch0 + ch1) * CE
    @functools.partial(
        pl.kernel,
        out_type=jax.ShapeDtypeStruct((tot, D), dtype),
        mesh=_mesh,
        scratch_types=[
            pltpu.VMEM((r0,), jnp.int32),
            pltpu.VMEM((CE, D), dtype),
            pltpu.VMEM((CE, D), dtype),
            pltpu.VMEM((CE, D), dtype),
            pltpu.SemaphoreType.DMA,
            pltpu.SemaphoreType.DMA,
            pltpu.SemaphoreType.DMA,
            pltpu.SemaphoreType.DMA,
            pltpu.SemaphoreType.DMA,
            pltpu.SemaphoreType.DMA,
        ],
    )
    def body(table_hbm, idx_hbm, out_hbm, idx_v, rows0, rows1, rows2,
             sg0, sg1, sg2, sw0, sw1, sw2):
        cid = lax.axis_index("c")
        pairbase = lax.axis_index("s") * rp
        rows = (rows0, rows1, rows2)
        sg = (sg0, sg1, sg2)
        sw = (sw0, sw1, sw2)
        nbuf = 3

        def pipeline(base0, nch):
            pltpu.sync_copy(idx_hbm.at[pl.ds(base0, nch * CE)],
                            idx_v.at[pl.ds(0, nch * CE)])

            def gather(c):
                b = c % nbuf
                return pltpu.make_async_copy(
                    table_hbm.at[idx_v.at[pl.ds(c * CE, CE)]], rows[b], sg[b])

            def write(c):
                b = c % nbuf
                return pltpu.make_async_copy(
                    rows[b], out_hbm.at[pl.ds(base0 + c * CE, CE)], sw[b])

            for b in range(min(nbuf - 1, nch)):
                gather(b).start()
            for c in range(nch):
                gather(c).wait()
                if c >= 1:
                    write(c - 1).wait()
                if c + nbuf - 1 < nch:
                    gather(c + nbuf - 1).start()
                write(c).start()
            write(nch - 1).wait()

        if ch0 > 0:
            pl.when(cid == 0)(lambda: pipeline(pairbase, ch0))
        if ch1 > 0:
            pl.when(cid != 0)(lambda: pipeline(pairbase + r0 * 0 + ch0 * CE, ch1))

    return body


_sc_gather_x_rows = _make_row_gather(jnp.float32, 60, 20)
_sc_gather_msg_half = _make_row_gather(jnp.float32, 29, 11, TOT // 2)
_sc_gather_tab_half = _make_row_gather(jnp.float32, 31, 9, TOT // 2)


DST_C0 = 53
DST_C1 = 27


@functools.partial(
    pl.kernel,
    out_type=jax.ShapeDtypeStruct((TOT,), jnp.int32),
    mesh=_mesh,
    scratch_types=[
        pltpu.VMEM((DST_C0 * CE,), jnp.int32),
        pltpu.VMEM((DST_C0 * CE,), jnp.int32),
        pltpu.VMEM((CE, D), jnp.int32),
        pltpu.VMEM((CE, D), jnp.int32),
        pltpu.VMEM((CE,), jnp.int32),
        pltpu.VMEM((CE,), jnp.int32),
        pltpu.SemaphoreType.DMA,
        pltpu.SemaphoreType.DMA,
        pltpu.SemaphoreType.DMA,
        pltpu.SemaphoreType.DMA,
    ],
    compiler_params=_no_layout,
)
def _sc_gather_dst(dstrows_hbm, nbr_hbm, dstg_hbm, idx_v, ridx_v,
                   drows0, drows1, didx0, didx1, sg0, sg1, sw0, sw1):
    cid = lax.axis_index("c")
    pairbase = lax.axis_index("s") * RP
    drows = (drows0, drows1)
    didx = (didx0, didx1)
    sg = (sg0, sg1)
    sw = (sw0, sw1)

    def pipeline(base0, nch):
        pltpu.sync_copy(nbr_hbm.at[pl.ds(base0, nch * CE)],
                        idx_v.at[pl.ds(0, nch * CE)])

        @pl.loop(0, nch * CE // 16)
        def _(j):
            sl = pl.ds(j * 16, 16)
            ridx_v[sl] = (lax.shift_right_logical(idx_v[sl], 7)
                          + lax.iota(jnp.int32, 16) * (E // D))

        def gather(c):
            b = c % 2
            return pltpu.make_async_copy(
                dstrows_hbm.at[ridx_v.at[pl.ds(c * CE, CE)]], drows[b], sg[b])

        def write(c):
            b = c % 2
            return pltpu.make_async_copy(
                didx[b], dstg_hbm.at[pl.ds(base0 + c * CE, CE)], sw[b])

        gather(0).start()
        for c in range(nch):
            b = c % 2
            gather(c).wait()
            if c >= 1:
                write(c - 1).wait()
            if c + 1 < nch:
                gather(c + 1).start()

            @pl.loop(0, CE // 16)
            def _(j, b=b, c=c):
                sl = pl.ds(j * 16, 16)
                r = lax.iota(jnp.int32, 16) + j * 16
                col = lax.bitwise_and(idx_v[pl.ds(c * CE + j * 16, 16)],
                                      jnp.full((16,), 127, jnp.int32))
                didx[b][sl] = plsc.load_gather(drows[b], [r, col])

            write(c).start()
        write(nch - 1).wait()

    pl.when(cid == 0)(lambda: pipeline(pairbase, DST_C0))
    pl.when(cid != 0)(lambda: pipeline(pairbase + DST_C0 * CE, DST_C1))


def _msg_body(xg_ref, eap_ref, wpad_ref, msg_ref):
    ea = jnp.dot(eap_ref[...], wpad_ref[...],
                 preferred_element_type=jnp.float32)
    msg_ref[...] = jax.nn.relu(xg_ref[...] + ea) + 1e-07


_EB = 4096


def _tc_msg(xg, eap, wpad):
    return pl.pallas_call(
        _msg_body,
        grid=(TOT // _EB,),
        in_specs=[
            pl.BlockSpec((_EB, D), lambda i: (i, 0)),
            pl.BlockSpec((_EB, D), lambda i: (i, 0)),
            pl.BlockSpec((D, D), lambda i: (0, 0)),
        ],
        out_specs=pl.BlockSpec((_EB, D), lambda i: (i, 0)),
        out_shape=jax.ShapeDtypeStruct((TOT, D), jnp.float32),
    )(xg, eap, wpad)


_NB = 128


def _max_body(mg_ref, sm_ref):
    sm_ref[...] = jnp.max(mg_ref[...], axis=1)


def _tc_max(mg3):
    n = mg3.shape[0]
    return pl.pallas_call(
        _max_body,
        grid=(n // _NB,),
        in_specs=[pl.BlockSpec((_NB, K, D), lambda i: (i, 0, 0))],
        out_specs=pl.BlockSpec((_NB, D), lambda i: (i, 0)),
        out_shape=jax.ShapeDtypeStruct((n, D), jnp.float32),
    )(mg3)


def _exp_body(mg_ref, smg_ref, os_ref, u_ref):
    m = mg_ref[...]
    t = jnp.exp(m - smg_ref[...])
    u_ref[...] = m * t
    os_ref[...] = jnp.sum(t, axis=1) + 1e-16


def _tc_expsum(mg3, smg3):
    n = mg3.shape[0]
    return pl.pallas_call(
        _exp_body,
        grid=(n // _NB,),
        in_specs=[
            pl.BlockSpec((_NB, K, D), lambda i: (i, 0, 0)),
            pl.BlockSpec((_NB, K, D), lambda i: (i, 0, 0)),
        ],
        out_specs=[
            pl.BlockSpec((_NB, D), lambda i: (i, 0)),
            pl.BlockSpec((_NB, K, D), lambda i: (i, 0, 0)),
        ],
        out_shape=[
            jax.ShapeDtypeStruct((n, D), jnp.float32),
            jax.ShapeDtypeStruct((n, K, D), jnp.float32),
        ],
    )(mg3, smg3)


def _agg_body(u_ref, osg_ref, agg_ref):
    agg_ref[...] = jnp.sum(u_ref[...] / osg_ref[...], axis=1)


def _tc_agg(u3, osg3):
    n = u3.shape[0]
    return pl.pallas_call(
        _agg_body,
        grid=(n // _NB,),
        in_specs=[
            pl.BlockSpec((_NB, K, D), lambda i: (i, 0, 0)),
            pl.BlockSpec((_NB, K, D), lambda i: (i, 0, 0)),
        ],
        out_specs=pl.BlockSpec((_NB, D), lambda i: (i, 0)),
        out_shape=jax.ShapeDtypeStruct((n, D), jnp.float32),
    )(u3, osg3)


def _mlp_body(agg_ref, x_ref, w1t_ref, gamma_ref, beta_ref, w2t_ref, out_ref):
    out = agg_ref[...] + x_ref[...]
    h = jnp.dot(out, w1t_ref[...], preferred_element_type=jnp.float32)
    mean = jnp.mean(h, axis=0)
    var = jnp.mean((h - mean[None, :]) ** 2, axis=0)
    h = (h - mean[None, :]) / jnp.sqrt(var[None, :] + 1e-05)
    h = h * gamma_ref[...][None, :] + beta_ref[...][None, :]
    h = jax.nn.relu(h)
    out_ref[...] = jnp.dot(h, w2t_ref[...], preferred_element_type=jnp.float32)


def _tc_mlp(agg, x, w1t, gamma, beta, w2t):
    return pl.pallas_call(
        _mlp_body,
        out_shape=jax.ShapeDtypeStruct((N, D), jnp.float32),
    )(agg, x, w1t, gamma, beta, w2t)


def kernel(x, edge_index, edge_attr, nbr, W_edge, W1, gamma, beta, W2):
    src = edge_index[0]
    dst = edge_index[1]

    src_p = jnp.concatenate([src, jnp.zeros((TOT - E,), jnp.int32)])
    eap = jnp.pad(edge_attr, ((0, TOT - E), (0, D - ED))).astype(jnp.bfloat16)
    dstrows = dst.reshape(E // D, D)
    nbr_p = jnp.concatenate(
        [nbr, jnp.zeros((NP - N, K), jnp.int32)], axis=0)
    nbrf = nbr_p.reshape(-1)

    wpad = jnp.pad(W_edge.T, ((0, D - ED), (0, 0))).astype(jnp.bfloat16)
    w1t = W1.T
    w2t = W2.T

    rep = (jnp.arange(TOT, dtype=jnp.int32) & 7)
    x8 = jnp.tile(x, (8, 1))
    xg = _sc_gather_x_rows(x8, src_p + rep * N)

    dstg = _sc_gather_dst(jnp.tile(dstrows, (16, 1)), nbrf)
    dstg_r = dstg + rep * NP

    msg = _tc_msg(xg, eap, wpad)

    H = TOT // 2
    NH = NP // 2
    nbrf_h = (nbrf[:H], nbrf[H:])
    dstg_h = (dstg_r[:H], dstg_r[H:])

    mg = [_sc_gather_msg_half(msg, i) for i in nbrf_h]
    mg3 = [m.reshape(NH, K, D) for m in mg]
    sm = [_tc_max(m) for m in mg3]

    sm8 = jnp.tile(jnp.concatenate(sm), (8, 1))
    smg = [_sc_gather_tab_half(sm8, i) for i in dstg_h]
    ou = [_tc_expsum(m, s.reshape(NH, K, D)) for m, s in zip(mg3, smg)]

    os8 = jnp.tile(jnp.concatenate([o for o, _ in ou]), (8, 1))
    osg = [_sc_gather_tab_half(os8, i) for i in dstg_h]
    agg = [_tc_agg(u, o.reshape(NH, K, D)) for (_, u), o in zip(ou, osg)]

    agg_all = jnp.concatenate(agg)
    return _tc_mlp(agg_all[:N], x, w1t, gamma, beta, w2t)


# speedup vs baseline: 1.0574x; 1.0574x over previous
import dataclasses
import functools

import jax
import jax.numpy as jnp
from jax import lax
from jax.experimental import pallas as pl
from jax.experimental.pallas import tpu as pltpu
from jax.experimental.pallas import tpu_sc as plsc

N = 10000
E = 320000
D = 128
ED = 16
K = 32

NC = 2
NS = 16
NW = NC * NS

PER_W = 10240
TOT = NW * PER_W
NP = TOT // K

CE = 256

CH_C0 = 60
CH_C1 = 20
R0 = CH_C0 * CE
R1 = CH_C1 * CE
RP = R0 + R1

_mesh = plsc.VectorSubcoreMesh(core_axis_name="c", subcore_axis_name="s")

_no_layout = pltpu.CompilerParams()
if "needs_layout_passes" in pltpu.CompilerParams.__dataclass_fields__:
    _no_layout = dataclasses.replace(_no_layout, needs_layout_passes=False)


def _wid():
    return lax.axis_index("s") * NC + lax.axis_index("c")


def _make_row_gather(dtype, ch0=CH_C0, ch1=CH_C1, tot=TOT):
    r0 = ch0 * CE
    rp = (ch0 + ch1) * CE
    @functools.partial(
        pl.kernel,
        out_type=jax.ShapeDtypeStruct((tot, D), dtype),
        mesh=_mesh,
        scratch_types=[
            pltpu.VMEM((r0,), jnp.int32),
            pltpu.VMEM((CE, D), dtype),
            pltpu.VMEM((CE, D), dtype),
            pltpu.VMEM((CE, D), dtype),
            pltpu.SemaphoreType.DMA,
            pltpu.SemaphoreType.DMA,
            pltpu.SemaphoreType.DMA,
            pltpu.SemaphoreType.DMA,
            pltpu.SemaphoreType.DMA,
            pltpu.SemaphoreType.DMA,
        ],
    )
    def body(table_hbm, idx_hbm, out_hbm, idx_v, rows0, rows1, rows2,
             sg0, sg1, sg2, sw0, sw1, sw2):
        cid = lax.axis_index("c")
        pairbase = lax.axis_index("s") * rp
        rows = (rows0, rows1, rows2)
        sg = (sg0, sg1, sg2)
        sw = (sw0, sw1, sw2)
        nbuf = 3

        def pipeline(base0, nch):
            pltpu.sync_copy(idx_hbm.at[pl.ds(base0, nch * CE)],
                            idx_v.at[pl.ds(0, nch * CE)])

            def gather(c):
                b = c % nbuf
                return pltpu.make_async_copy(
                    table_hbm.at[idx_v.at[pl.ds(c * CE, CE)]], rows[b], sg[b])

            def write(c):
                b = c % nbuf
                return pltpu.make_async_copy(
                    rows[b], out_hbm.at[pl.ds(base0 + c * CE, CE)], sw[b])

            for b in range(min(nbuf - 1, nch)):
                gather(b).start()
            for c in range(nch):
                gather(c).wait()
                if c >= 1:
                    write(c - 1).wait()
                if c + nbuf - 1 < nch:
                    gather(c + nbuf - 1).start()
                write(c).start()
            write(nch - 1).wait()

        if ch0 > 0:
            pl.when(cid == 0)(lambda: pipeline(pairbase, ch0))
        if ch1 > 0:
            pl.when(cid != 0)(lambda: pipeline(pairbase + r0 * 0 + ch0 * CE, ch1))

    return body


_sc_gather_x_rows = _make_row_gather(jnp.float32, 60, 20)
_sc_gather_msg_rows = _make_row_gather(jnp.float32, 57, 23)
_sc_gather_tab_rows = _make_row_gather(jnp.float32, 63, 17)


DST_C0 = 53
DST_C1 = 27


@functools.partial(
    pl.kernel,
    out_type=jax.ShapeDtypeStruct((TOT,), jnp.int32),
    mesh=_mesh,
    scratch_types=[
        pltpu.VMEM((DST_C0 * CE,), jnp.int32),
        pltpu.VMEM((DST_C0 * CE,), jnp.int32),
        pltpu.VMEM((CE, D), jnp.int32),
        pltpu.VMEM((CE, D), jnp.int32),
        pltpu.VMEM((CE,), jnp.int32),
        pltpu.VMEM((CE,), jnp.int32),
        pltpu.SemaphoreType.DMA,
        pltpu.SemaphoreType.DMA,
        pltpu.SemaphoreType.DMA,
        pltpu.SemaphoreType.DMA,
    ],
    compiler_params=_no_layout,
)
def _sc_gather_dst(dstrows_hbm, nbr_hbm, dstg_hbm, idx_v, ridx_v,
                   drows0, drows1, didx0, didx1, sg0, sg1, sw0, sw1):
    cid = lax.axis_index("c")
    pairbase = lax.axis_index("s") * RP
    drows = (drows0, drows1)
    didx = (didx0, didx1)
    sg = (sg0, sg1)
    sw = (sw0, sw1)

    def pipeline(base0, nch):
        pltpu.sync_copy(nbr_hbm.at[pl.ds(base0, nch * CE)],
                        idx_v.at[pl.ds(0, nch * CE)])

        @pl.loop(0, nch * CE // 16)
        def _(j):
            sl = pl.ds(j * 16, 16)
            ridx_v[sl] = (lax.shift_right_logical(idx_v[sl], 7)
                          + lax.iota(jnp.int32, 16) * (E // D))

        def gather(c):
            b = c % 2
            return pltpu.make_async_copy(
                dstrows_hbm.at[ridx_v.at[pl.ds(c * CE, CE)]], drows[b], sg[b])

        def write(c):
            b = c % 2
            return pltpu.make_async_copy(
                didx[b], dstg_hbm.at[pl.ds(base0 + c * CE, CE)], sw[b])

        gather(0).start()
        for c in range(nch):
            b = c % 2
            gather(c).wait()
            if c >= 1:
                write(c - 1).wait()
            if c + 1 < nch:
                gather(c + 1).start()

            @pl.loop(0, CE // 16)
            def _(j, b=b, c=c):
                sl = pl.ds(j * 16, 16)
                r = lax.iota(jnp.int32, 16) + j * 16
                col = lax.bitwise_and(idx_v[pl.ds(c * CE + j * 16, 16)],
                                      jnp.full((16,), 127, jnp.int32))
                didx[b][sl] = plsc.load_gather(drows[b], [r, col])

            write(c).start()
        write(nch - 1).wait()

    pl.when(cid == 0)(lambda: pipeline(pairbase, DST_C0))
    pl.when(cid != 0)(lambda: pipeline(pairbase + DST_C0 * CE, DST_C1))


def _msg_body(xg_ref, eap_ref, wpad_ref, msg_ref):
    ea = jnp.dot(eap_ref[...], wpad_ref[...],
                 preferred_element_type=jnp.float32)
    msg_ref[...] = jax.nn.relu(xg_ref[...] + ea) + 1e-07


_EB = 4096


def _tc_msg(xg, eap, wpad):
    return pl.pallas_call(
        _msg_body,
        grid=(TOT // _EB,),
        in_specs=[
            pl.BlockSpec((_EB, D), lambda i: (i, 0)),
            pl.BlockSpec((_EB, D), lambda i: (i, 0)),
            pl.BlockSpec((D, D), lambda i: (0, 0)),
        ],
        out_specs=pl.BlockSpec((_EB, D), lambda i: (i, 0)),
        out_shape=jax.ShapeDtypeStruct((TOT, D), jnp.float32),
    )(xg, eap, wpad)


_NB = 128


def _max_body(mg_ref, sm_ref):
    sm_ref[...] = jnp.max(mg_ref[...], axis=1)


def _tc_max(mg3):
    n = mg3.shape[0]
    return pl.pallas_call(
        _max_body,
        grid=(n // _NB,),
        in_specs=[pl.BlockSpec((_NB, K, D), lambda i: (i, 0, 0))],
        out_specs=pl.BlockSpec((_NB, D), lambda i: (i, 0)),
        out_shape=jax.ShapeDtypeStruct((n, D), jnp.float32),
    )(mg3)


def _exp_body(mg_ref, smg_ref, os_ref, u_ref):
    m = mg_ref[...]
    t = jnp.exp(m - smg_ref[...])
    u_ref[...] = m * t
    os_ref[...] = jnp.sum(t, axis=1) + 1e-16


def _tc_expsum(mg3, smg3):
    n = mg3.shape[0]
    return pl.pallas_call(
        _exp_body,
        grid=(n // _NB,),
        in_specs=[
            pl.BlockSpec((_NB, K, D), lambda i: (i, 0, 0)),
            pl.BlockSpec((_NB, K, D), lambda i: (i, 0, 0)),
        ],
        out_specs=[
            pl.BlockSpec((_NB, D), lambda i: (i, 0)),
            pl.BlockSpec((_NB, K, D), lambda i: (i, 0, 0)),
        ],
        out_shape=[
            jax.ShapeDtypeStruct((n, D), jnp.float32),
            jax.ShapeDtypeStruct((n, K, D), jnp.float32),
        ],
    )(mg3, smg3)


def _agg_body(u_ref, osg_ref, agg_ref):
    agg_ref[...] = jnp.sum(u_ref[...] / osg_ref[...], axis=1)


def _tc_agg(u3, osg3):
    n = u3.shape[0]
    return pl.pallas_call(
        _agg_body,
        grid=(n // _NB,),
        in_specs=[
            pl.BlockSpec((_NB, K, D), lambda i: (i, 0, 0)),
            pl.BlockSpec((_NB, K, D), lambda i: (i, 0, 0)),
        ],
        out_specs=pl.BlockSpec((_NB, D), lambda i: (i, 0)),
        out_shape=jax.ShapeDtypeStruct((n, D), jnp.float32),
    )(u3, osg3)


def _mlp_body(agg_ref, x_ref, w1t_ref, gamma_ref, beta_ref, w2t_ref, out_ref):
    out = agg_ref[...] + x_ref[...]
    h = jnp.dot(out, w1t_ref[...], preferred_element_type=jnp.float32)
    mean = jnp.mean(h, axis=0)
    var = jnp.mean((h - mean[None, :]) ** 2, axis=0)
    h = (h - mean[None, :]) / jnp.sqrt(var[None, :] + 1e-05)
    h = h * gamma_ref[...][None, :] + beta_ref[...][None, :]
    h = jax.nn.relu(h)
    out_ref[...] = jnp.dot(h, w2t_ref[...], preferred_element_type=jnp.float32)


def _tc_mlp(agg, x, w1t, gamma, beta, w2t):
    return pl.pallas_call(
        _mlp_body,
        out_shape=jax.ShapeDtypeStruct((N, D), jnp.float32),
    )(agg, x, w1t, gamma, beta, w2t)


def kernel(x, edge_index, edge_attr, nbr, W_edge, W1, gamma, beta, W2):
    src = edge_index[0]
    dst = edge_index[1]

    src_p = jnp.concatenate([src, jnp.zeros((TOT - E,), jnp.int32)])
    eap = jnp.pad(edge_attr, ((0, TOT - E), (0, D - ED))).astype(jnp.bfloat16)
    dstrows = dst.reshape(E // D, D)
    nbr_p = jnp.concatenate(
        [nbr, jnp.zeros((NP - N, K), jnp.int32)], axis=0)
    nbrf = nbr_p.reshape(-1)

    wpad = jnp.pad(W_edge.T, ((0, D - ED), (0, 0))).astype(jnp.bfloat16)
    w1t = W1.T
    w2t = W2.T

    rep = (jnp.arange(TOT, dtype=jnp.int32) & 7)
    x8 = jnp.tile(x, (8, 1))
    xg = _sc_gather_x_rows(x8, src_p + rep * N)

    dstg = _sc_gather_dst(jnp.tile(dstrows, (16, 1)), nbrf)
    dstg_r = dstg + rep * NP

    msg = _tc_msg(xg, eap, wpad)

    mg = _sc_gather_msg_rows(msg, nbrf)
    mg3 = mg.reshape(NP, K, D)

    sm = _tc_max(mg3)
    smg = _sc_gather_tab_rows(jnp.tile(sm, (8, 1)), dstg_r)

    osum, u = _tc_expsum(mg3, smg.reshape(NP, K, D))
    osg = _sc_gather_tab_rows(jnp.tile(osum, (8, 1)), dstg_r)

    agg = _tc_agg(u, osg.reshape(NP, K, D))

    return _tc_mlp(agg[:N], x, w1t, gamma, beta, w2t)


# speedup vs baseline: 1.0700x; 1.0119x over previous
import dataclasses
import functools

import jax
import jax.numpy as jnp
from jax import lax
from jax.experimental import pallas as pl
from jax.experimental.pallas import tpu as pltpu
from jax.experimental.pallas import tpu_sc as plsc

N = 10000
E = 320000
D = 128
ED = 16
K = 32

NC = 2
NS = 16
NW = NC * NS

PER_W = 10240
TOT = NW * PER_W
NP = TOT // K

CE = 256

CH_C0 = 60
CH_C1 = 20
R0 = CH_C0 * CE
R1 = CH_C1 * CE
RP = R0 + R1

_mesh = plsc.VectorSubcoreMesh(core_axis_name="c", subcore_axis_name="s")

_no_layout = pltpu.CompilerParams()
if "needs_layout_passes" in pltpu.CompilerParams.__dataclass_fields__:
    _no_layout = dataclasses.replace(_no_layout, needs_layout_passes=False)


def _wid():
    return lax.axis_index("s") * NC + lax.axis_index("c")


def _make_row_gather(dtype, ch0=CH_C0, ch1=CH_C1, tot=TOT):
    r0 = ch0 * CE
    rp = (ch0 + ch1) * CE
    @functools.partial(
        pl.kernel,
        out_type=jax.ShapeDtypeStruct((tot, D), dtype),
        mesh=_mesh,
        scratch_types=[
            pltpu.VMEM((r0,), jnp.int32),
            pltpu.VMEM((CE, D), dtype),
            pltpu.VMEM((CE, D), dtype),
            pltpu.VMEM((CE, D), dtype),
            pltpu.SemaphoreType.DMA,
            pltpu.SemaphoreType.DMA,
            pltpu.SemaphoreType.DMA,
            pltpu.SemaphoreType.DMA,
            pltpu.SemaphoreType.DMA,
            pltpu.SemaphoreType.DMA,
        ],
    )
    def body(table_hbm, idx_hbm, out_hbm, idx_v, rows0, rows1, rows2,
             sg0, sg1, sg2, sw0, sw1, sw2):
        cid = lax.axis_index("c")
        pairbase = lax.axis_index("s") * rp
        rows = (rows0, rows1, rows2)
        sg = (sg0, sg1, sg2)
        sw = (sw0, sw1, sw2)
        nbuf = 3

        def pipeline(base0, nch):
            pltpu.sync_copy(idx_hbm.at[pl.ds(base0, nch * CE)],
                            idx_v.at[pl.ds(0, nch * CE)])

            def gather(c):
                b = c % nbuf
                return pltpu.make_async_copy(
                    table_hbm.at[idx_v.at[pl.ds(c * CE, CE)]], rows[b], sg[b])

            def write(c):
                b = c % nbuf
                return pltpu.make_async_copy(
                    rows[b], out_hbm.at[pl.ds(base0 + c * CE, CE)], sw[b])

            for b in range(min(nbuf - 1, nch)):
                gather(b).start()
            for c in range(nch):
                gather(c).wait()
                if c >= 1:
                    write(c - 1).wait()
                if c + nbuf - 1 < nch:
                    gather(c + nbuf - 1).start()
                write(c).start()
            write(nch - 1).wait()

        if ch0 > 0:
            pl.when(cid == 0)(lambda: pipeline(pairbase, ch0))
        if ch1 > 0:
            pl.when(cid != 0)(lambda: pipeline(pairbase + r0 * 0 + ch0 * CE, ch1))

    return body


_sc_gather_x_rows = _make_row_gather(jnp.float32, 60, 20)
_sc_gather_msg_rows = _make_row_gather(jnp.float32, 57, 23)
_sc_gather_tab_rows = _make_row_gather(jnp.float32, 63, 17)


DST_C0 = 53
DST_C1 = 27


@functools.partial(
    pl.kernel,
    out_type=jax.ShapeDtypeStruct((TOT,), jnp.int32),
    mesh=_mesh,
    scratch_types=[
        pltpu.VMEM((DST_C0 * CE,), jnp.int32),
        pltpu.VMEM((DST_C0 * CE,), jnp.int32),
        pltpu.VMEM((CE, D), jnp.int32),
        pltpu.VMEM((CE, D), jnp.int32),
        pltpu.VMEM((CE,), jnp.int32),
        pltpu.VMEM((CE,), jnp.int32),
        pltpu.SemaphoreType.DMA,
        pltpu.SemaphoreType.DMA,
        pltpu.SemaphoreType.DMA,
        pltpu.SemaphoreType.DMA,
    ],
    compiler_params=_no_layout,
)
def _sc_gather_dst(dstrows_hbm, nbr_hbm, dstg_hbm, idx_v, ridx_v,
                   drows0, drows1, didx0, didx1, sg0, sg1, sw0, sw1):
    cid = lax.axis_index("c")
    pairbase = lax.axis_index("s") * RP
    drows = (drows0, drows1)
    didx = (didx0, didx1)
    sg = (sg0, sg1)
    sw = (sw0, sw1)

    def pipeline(base0, nch):
        pltpu.sync_copy(nbr_hbm.at[pl.ds(base0, nch * CE)],
                        idx_v.at[pl.ds(0, nch * CE)])

        @pl.loop(0, nch * CE // 16)
        def _(j):
            sl = pl.ds(j * 16, 16)
            ridx_v[sl] = (lax.shift_right_logical(idx_v[sl], 7)
                          + lax.iota(jnp.int32, 16) * (E // D))

        def gather(c):
            b = c % 2
            return pltpu.make_async_copy(
                dstrows_hbm.at[ridx_v.at[pl.ds(c * CE, CE)]], drows[b], sg[b])

        def write(c):
            b = c % 2
            return pltpu.make_async_copy(
                didx[b], dstg_hbm.at[pl.ds(base0 + c * CE, CE)], sw[b])

        gather(0).start()
        for c in range(nch):
            b = c % 2
            gather(c).wait()
            if c >= 1:
                write(c - 1).wait()
            if c + 1 < nch:
                gather(c + 1).start()

            @pl.loop(0, CE // 16)
            def _(j, b=b, c=c):
                sl = pl.ds(j * 16, 16)
                r = lax.iota(jnp.int32, 16) + j * 16
                col = lax.bitwise_and(idx_v[pl.ds(c * CE + j * 16, 16)],
                                      jnp.full((16,), 127, jnp.int32))
                didx[b][sl] = plsc.load_gather(drows[b], [r, col])

            write(c).start()
        write(nch - 1).wait()

    pl.when(cid == 0)(lambda: pipeline(pairbase, DST_C0))
    pl.when(cid != 0)(lambda: pipeline(pairbase + DST_C0 * CE, DST_C1))


def _msg_body(xg_ref, eap_ref, wpad_ref, msg_ref):
    ea = jnp.dot(eap_ref[...], wpad_ref[...],
                 preferred_element_type=jnp.float32)
    msg_ref[...] = jax.nn.relu(xg_ref[...] + ea) + 1e-07


_EB = 4096


def _tc_msg(xg, eap, wpad):
    return pl.pallas_call(
        _msg_body,
        grid=(TOT // _EB,),
        in_specs=[
            pl.BlockSpec((_EB, D), lambda i: (i, 0)),
            pl.BlockSpec((_EB, D), lambda i: (i, 0)),
            pl.BlockSpec((D, D), lambda i: (0, 0)),
        ],
        out_specs=pl.BlockSpec((_EB, D), lambda i: (i, 0)),
        out_shape=jax.ShapeDtypeStruct((TOT, D), jnp.float32),
    )(xg, eap, wpad)


_NB = 128


def _max_body(mg_ref, sm_ref):
    sm_ref[...] = jnp.max(mg_ref[...], axis=1)


def _tc_max(mg3):
    n = mg3.shape[0]
    return pl.pallas_call(
        _max_body,
        grid=(n // _NB,),
        in_specs=[pl.BlockSpec((_NB, K, D), lambda i: (i, 0, 0))],
        out_specs=pl.BlockSpec((_NB, D), lambda i: (i, 0)),
        out_shape=jax.ShapeDtypeStruct((n, D), jnp.float32),
    )(mg3)


def _exp_body(mg_ref, smg_ref, os_ref, u_ref):
    m = mg_ref[...]
    t = jnp.exp(m - smg_ref[...])
    u_ref[...] = m * t
    os_ref[...] = jnp.sum(t, axis=1) + 1e-16


def _tc_expsum(mg3, smg3):
    n = mg3.shape[0]
    return pl.pallas_call(
        _exp_body,
        grid=(n // _NB,),
        in_specs=[
            pl.BlockSpec((_NB, K, D), lambda i: (i, 0, 0)),
            pl.BlockSpec((_NB, K, D), lambda i: (i, 0, 0)),
        ],
        out_specs=[
            pl.BlockSpec((_NB, D), lambda i: (i, 0)),
            pl.BlockSpec((_NB, K, D), lambda i: (i, 0, 0)),
        ],
        out_shape=[
            jax.ShapeDtypeStruct((n, D), jnp.float32),
            jax.ShapeDtypeStruct((n, K, D), jnp.float32),
        ],
    )(mg3, smg3)


def _agg_body(u_ref, osg_ref, agg_ref):
    agg_ref[...] = jnp.sum(u_ref[...] / osg_ref[...], axis=1)


def _tc_agg(u3, osg3):
    n = u3.shape[0]
    return pl.pallas_call(
        _agg_body,
        grid=(n // _NB,),
        in_specs=[
            pl.BlockSpec((_NB, K, D), lambda i: (i, 0, 0)),
            pl.BlockSpec((_NB, K, D), lambda i: (i, 0, 0)),
        ],
        out_specs=pl.BlockSpec((_NB, D), lambda i: (i, 0)),
        out_shape=jax.ShapeDtypeStruct((n, D), jnp.float32),
    )(u3, osg3)


def _mlp_body(agg_ref, x_ref, w1t_ref, gamma_ref, beta_ref, w2t_ref, out_ref):
    out = agg_ref[...] + x_ref[...]
    h = jnp.dot(out, w1t_ref[...], preferred_element_type=jnp.float32)
    mean = jnp.mean(h, axis=0)
    var = jnp.mean((h - mean[None, :]) ** 2, axis=0)
    h = (h - mean[None, :]) / jnp.sqrt(var[None, :] + 1e-05)
    h = h * gamma_ref[...][None, :] + beta_ref[...][None, :]
    h = jax.nn.relu(h)
    out_ref[...] = jnp.dot(h, w2t_ref[...], preferred_element_type=jnp.float32)


def _tc_mlp(agg, x, w1t, gamma, beta, w2t):
    return pl.pallas_call(
        _mlp_body,
        out_shape=jax.ShapeDtypeStruct((N, D), jnp.float32),
    )(agg, x, w1t, gamma, beta, w2t)


def kernel(x, edge_index, edge_attr, nbr, W_edge, W1, gamma, beta, W2):
    src = edge_index[0]
    dst = edge_index[1]

    src_p = jnp.concatenate([src, jnp.zeros((TOT - E,), jnp.int32)])
    eap = jnp.pad(edge_attr, ((0, TOT - E), (0, D - ED))).astype(jnp.bfloat16)
    dstrows = dst.reshape(E // D, D)
    nbr_p = jnp.concatenate(
        [nbr, jnp.zeros((NP - N, K), jnp.int32)], axis=0)
    nbrf = nbr_p.reshape(-1)

    wpad = jnp.pad(W_edge.T, ((0, D - ED), (0, 0))).astype(jnp.bfloat16)
    w1t = W1.T
    w2t = W2.T

    rep = (jnp.arange(TOT, dtype=jnp.int32) & 15)
    x16 = jnp.tile(x, (16, 1))
    xg = _sc_gather_x_rows(x16, src_p + rep * N)

    dstg = _sc_gather_dst(jnp.tile(dstrows, (16, 1)), nbrf)
    dstg_r = dstg + rep * NP

    msg = _tc_msg(xg, eap, wpad)

    mg = _sc_gather_msg_rows(msg, nbrf)
    mg3 = mg.reshape(NP, K, D)

    sm = _tc_max(mg3)
    smg = _sc_gather_tab_rows(jnp.tile(sm, (16, 1)), dstg_r)

    osum, u = _tc_expsum(mg3, smg.reshape(NP, K, D))
    osg = _sc_gather_tab_rows(jnp.tile(osum, (16, 1)), dstg_r)

    agg = _tc_agg(u, osg.reshape(NP, K, D))

    return _tc_mlp(agg[:N], x, w1t, gamma, beta, w2t)
